# Initial kernel scaffold; baseline (speedup 1.0000x reference)
import jax
import jax.numpy as jnp
from jax.experimental import pallas as pl


def kernel(gaussian_centers, score_maps, extrinsics, intrinsics, alphas):
    raise NotImplementedError("write your pallas kernel here")


# speedup vs baseline: 78.1737x; 78.1737x over previous
import jax
import jax.numpy as jnp
from jax import lax
from jax.experimental import pallas as pl
from jax.experimental.pallas import tpu as pltpu
from jax.experimental.pallas import tpu_sc as plsc

N_PTS = 200000
NVIEW = 4
IMG_H = 256
IMG_W = 256
HW = IMG_H * IMG_W
NC = 2
NS = 16
NW = NC * NS
LANES = 16
CHUNK = 6272
NPAD = NW * CHUNK
ITERS = CHUNK // LANES
NCOEF = 19
EPS = 1e-8


def _bf16_round(v):
    r = lax.bitcast_convert_type(v, jnp.uint32)
    lsb = lax.shift_right_logical(r, jnp.uint32(16)) & jnp.uint32(1)
    r = (r + jnp.uint32(0x7FFF) + lsb) & jnp.uint32(0xFFFF0000)
    return lax.bitcast_convert_type(r, jnp.float32)


def _sc_body(coef_hbm, xs_hbm, ys_hbm, zs_hbm, maps_hbm, out_hbm,
             coef_v, x_v, y_v, z_v, map_v, acc_v):
    wid = lax.axis_index("s") * NC + lax.axis_index("c")
    base = wid * CHUNK
    pltpu.sync_copy(coef_hbm, coef_v)
    pltpu.sync_copy(xs_hbm.at[pl.ds(base, CHUNK)], x_v)
    pltpu.sync_copy(ys_hbm.at[pl.ds(base, CHUNK)], y_v)
    pltpu.sync_copy(zs_hbm.at[pl.ds(base, CHUNK)], z_v)

    for view in range(NVIEW):
        pltpu.sync_copy(maps_hbm.at[view], map_v)
        cv = [coef_v[pl.ds((view * NCOEF + j) * LANES, LANES)]
              for j in range(NCOEF)]
        (a0, a1, a2, a3, b0, b1, b2, b3, c0, c1, c2, c3,
         i00, i01, i02, i10, i11, i12, alpha) = cv

        def body(i, carry, view=view):
            s = i * LANES
            x = x_v[pl.ds(s, LANES)]
            y = y_v[pl.ds(s, LANES)]
            z = z_v[pl.ds(s, LANES)]
            camx = x * a0 + y * a1 + z * a2 + a3
            camy = x * b0 + y * b1 + z * b2 + b3
            camz = x * c0 + y * c1 + z * c2 + c3
            zd = camz + EPS
            xn = _bf16_round(camx / zd)
            yn = _bf16_round(camy / zd)
            u = xn * i00 + yn * i01 + i02
            v_ = xn * i10 + yn * i11 + i12
            zero = jnp.float32(0.0)
            one = jnp.float32(1.0)
            m = ((u >= zero) & (u < one) & (v_ >= zero) & (v_ < one)
                 & (camz > jnp.float32(EPS)))
            px = jnp.clip((u * jnp.float32(IMG_W)).astype(jnp.int32),
                          0, IMG_W - 1)
            py = jnp.clip((v_ * jnp.float32(IMG_H)).astype(jnp.int32),
                          0, IMG_H - 1)
            val = plsc.load_gather(map_v, [py, px])
            res = jnp.where(m, val, zero) * alpha
            if view == 0:
                acc_v[pl.ds(s, LANES)] = res
            else:
                acc_v[pl.ds(s, LANES)] = acc_v[pl.ds(s, LANES)] + res
            return carry

        lax.fori_loop(0, ITERS, body, 0)

    pltpu.sync_copy(acc_v, out_hbm.at[pl.ds(base, CHUNK)])


def kernel(gaussian_centers, score_maps, extrinsics, intrinsics, alphas):
    def bf(a):
        return a.astype(jnp.bfloat16).astype(jnp.float32)

    w2c = jnp.linalg.inv(extrinsics)
    intr = intrinsics.astype(jnp.float32)
    coef = jnp.concatenate(
        [bf(w2c[:, 0, :]), bf(w2c[:, 1, :]), bf(w2c[:, 2, :]),
         bf(intr[:, 0, :2]), intr[:, 0, 2:3],
         bf(intr[:, 1, :2]), intr[:, 1, 2:3],
         bf(alphas[:, None])], axis=1).astype(jnp.float32)
    coef_b = jnp.broadcast_to(coef[:, :, None],
                              (NVIEW, NCOEF, LANES)).reshape(-1)

    pts = jnp.concatenate(
        [bf(gaussian_centers.astype(jnp.float32)),
         jnp.zeros((NPAD - N_PTS, 3), jnp.float32)], axis=0)
    xs = pts[:, 0]
    ys = pts[:, 1]
    zs = pts[:, 2]
    maps_flat = bf(score_maps.astype(jnp.float32))

    mesh = plsc.VectorSubcoreMesh(core_axis_name="c", subcore_axis_name="s",
                                  num_cores=NC, num_subcores=NS)
    call = pl.kernel(
        _sc_body,
        out_type=jax.ShapeDtypeStruct((NPAD,), jnp.float32),
        mesh=mesh,
        compiler_params=pltpu.CompilerParams(use_tc_tiling_on_sc=False,
                                             needs_layout_passes=False),
        scratch_types=[
            pltpu.VMEM((NVIEW * NCOEF * LANES,), jnp.float32),
            pltpu.VMEM((CHUNK,), jnp.float32),
            pltpu.VMEM((CHUNK,), jnp.float32),
            pltpu.VMEM((CHUNK,), jnp.float32),
            pltpu.VMEM((IMG_H, IMG_W), jnp.float32),
            pltpu.VMEM((CHUNK,), jnp.float32),
        ],
    )
    out = call(coef_b, xs, ys, zs, maps_flat)
    return out[:N_PTS]
